# baseline (speedup 1.0000x reference)
import functools

import jax
import jax.numpy as jnp
from jax import lax
from jax.experimental import pallas as pl
from jax.experimental.pallas import tpu as pltpu
from jax.experimental.pallas import tpu_sc as plsc

N = 10000
NPAD = 10240
F_IN = 128
HID = 16
FW = 16
F_OUT = 8
PR = NPAD * FW // 128

NC = 2
NS = 16
NW = NC * NS
CB = 125
CH = 80
NB = 20
NR = CH // NB
RPT = NPAD // NS

_mesh = plsc.VectorSubcoreMesh(
    core_axis_name="c", subcore_axis_name="s", num_cores=NC, num_subcores=NS
)
_sc_params = pltpu.CompilerParams(use_tc_tiling_on_sc=False)


@functools.partial(
    pl.kernel,
    mesh=_mesh,
    out_type=jax.ShapeDtypeStruct((NC, NPAD, FW), jnp.float32),
    scratch_types=[
        pltpu.VMEM((CH, CB), jnp.int32),
        pltpu.VMEM((128,), jnp.float32),
        pltpu.VMEM((RPT,), jnp.float32),
        pltpu.VMEM((RPT, FW), jnp.float32),
        pltpu.VMEM_SHARED((NPAD,), jnp.float32),
        pltpu.SemaphoreType.DMA,
    ],
    compiler_params=_sc_params,
)
def _deg(ei_hbm, out_hbm, dst_v, ones_v, st_v, buf_v, acc, sem):
    c = lax.axis_index("c")
    s = lax.axis_index("s")
    wid = s * NC + c
    dslab = pltpu.async_copy(ei_hbm.at[1, wid], dst_v, sem)
    for i in range(8):
        ones_v[pl.ds(i * 16, 16)] = jnp.ones((16,), jnp.float32)
    for i in range(RPT // 16):
        st_v[pl.ds(i * 16, 16)] = jnp.zeros((16,), jnp.float32)
    pltpu.sync_copy(st_v, acc.at[pl.ds(s * RPT, RPT)])
    dslab.wait()
    plsc.subcore_barrier()

    ones_row = ones_v.at[pl.ds(0, CB)]

    def body(r, carry):
        descs = [
            pltpu.async_copy(ones_row, acc.at[dst_v.at[r * NB + b]], sem,
                             add=True)
            for b in range(NB)
        ]
        for d in descs:
            d.wait()
        return carry

    lax.fori_loop(0, CH // NB, body, 0)
    plsc.subcore_barrier()
    pltpu.sync_copy(acc.at[pl.ds(s * RPT, RPT)], st_v)

    def rep(g, carry):
        v = st_v[pl.ds(g * 16, 16)]
        for l in range(16):
            buf_v[g * 16 + l] = jnp.broadcast_to(v[l], (FW,))
        return carry

    lax.fori_loop(0, RPT // 16, rep, 0)
    pltpu.sync_copy(buf_v, out_hbm.at[c, pl.ds(s * RPT, RPT)])


@functools.partial(
    pl.kernel,
    mesh=_mesh,
    out_type=jax.ShapeDtypeStruct((NC, NPAD, FW), jnp.float32),
    scratch_types=[
        pltpu.VMEM((CH, CB), jnp.int32),
        pltpu.VMEM((CH, CB), jnp.int32),
        pltpu.VMEM((2 * NB, CB, FW), jnp.float32),
        pltpu.VMEM((RPT, FW), jnp.float32),
        pltpu.VMEM_SHARED((NPAD, FW), jnp.float32),
        pltpu.SemaphoreType.DMA,
        pltpu.SemaphoreType.DMA,
    ],
    compiler_params=_sc_params,
)
def _agg(tbl_hbm, ei_hbm, out_hbm,
         src_v, dst_v, rows_v, buf_v, acc, gsem, ssem):
    c = lax.axis_index("c")
    s = lax.axis_index("s")
    wid = s * NC + c
    sslab = pltpu.async_copy(ei_hbm.at[0, wid], src_v, gsem)
    dslab = pltpu.async_copy(ei_hbm.at[1, wid], dst_v, gsem)

    def zb(r, carry):
        buf_v[r] = jnp.zeros((FW,), jnp.float32)
        return carry

    lax.fori_loop(0, RPT, zb, 0)
    pltpu.sync_copy(buf_v, acc.at[pl.ds(s * RPT, RPT)])
    sslab.wait()
    dslab.wait()
    plsc.subcore_barrier()

    def gfire(r):
        bank = (r % 2) * NB
        j0 = r * NB
        return [
            pltpu.async_copy(tbl_hbm.at[src_v.at[j0 + b]],
                             rows_v.at[bank + b], gsem)
            for b in range(NB)
        ]

    gds = {0: gfire(0), 1: gfire(1)}
    for r in range(NR):
        bank = (r % 2) * NB
        j0 = r * NB
        sds = []
        for b in range(NB):
            gds[r][b].wait()
            sds.append(
                pltpu.async_copy(rows_v.at[bank + b], acc.at[dst_v.at[j0 + b]],
                                 ssem, add=True))
        for d in sds:
            d.wait()
        if r + 2 < NR:
            gds[r + 2] = gfire(r + 2)
    plsc.subcore_barrier()
    pltpu.sync_copy(acc.at[pl.ds(s * RPT, RPT)], buf_v)
    pltpu.sync_copy(buf_v, out_hbm.at[c, pl.ds(s * RPT, RPT)])


_BP = 128


def _k1_body(x_ref, w_ref, degp_ref, h_ref, dis_ref):
    deg = degp_ref[0] + degp_ref[1] + 1.0
    dis = lax.rsqrt(deg)
    h = jnp.dot(x_ref[...], w_ref[...], preferred_element_type=jnp.float32)
    h_ref[...] = h * dis
    dis_ref[...] = dis


def _k1(x_r, w_big, degp):
    return pl.pallas_call(
        _k1_body,
        grid=(PR // _BP,),
        in_specs=[
            pl.BlockSpec((_BP, F_IN * 8), lambda i: (i, 0)),
            pl.BlockSpec((F_IN * 8, 128), lambda i: (0, 0)),
            pl.BlockSpec((NC, _BP, 128), lambda i: (0, i, 0)),
        ],
        out_specs=[
            pl.BlockSpec((_BP, 128), lambda i: (i, 0)),
            pl.BlockSpec((_BP, 128), lambda i: (i, 0)),
        ],
        out_shape=[
            jax.ShapeDtypeStruct((PR, 128), jnp.float32),
            jax.ShapeDtypeStruct((PR, 128), jnp.float32),
        ],
    )(x_r, w_big, degp)


def _k3_body(a_ref, h1_ref, dis_ref, w_ref, b_ref, h_ref):
    dis = dis_ref[...]
    agg = a_ref[0] + a_ref[1] + h1_ref[...]
    z = jnp.maximum(dis * agg + b_ref[...], 0.0)
    h_ref[...] = dis * jnp.dot(z, w_ref[...],
                               preferred_element_type=jnp.float32)


def _k3(accp, h1p, disp, w2_big, b1rep):
    return pl.pallas_call(
        _k3_body,
        grid=(PR // _BP,),
        in_specs=[
            pl.BlockSpec((NC, _BP, 128), lambda i: (0, i, 0)),
            pl.BlockSpec((_BP, 128), lambda i: (i, 0)),
            pl.BlockSpec((_BP, 128), lambda i: (i, 0)),
            pl.BlockSpec((128, 128), lambda i: (0, 0)),
            pl.BlockSpec((1, 128), lambda i: (0, 0)),
        ],
        out_specs=pl.BlockSpec((_BP, 128), lambda i: (i, 0)),
        out_shape=jax.ShapeDtypeStruct((PR, 128), jnp.float32),
    )(accp, h1p, disp, w2_big, b1rep)


def _k5_body(a_ref, h2_ref, dis_ref, b_ref, o_ref):
    agg = a_ref[0] + a_ref[1] + h2_ref[...]
    o_ref[...] = dis_ref[...] * agg + b_ref[...]


def _k5(accp, h2p, disp, b2rep):
    return pl.pallas_call(
        _k5_body,
        grid=(PR // _BP,),
        in_specs=[
            pl.BlockSpec((NC, _BP, 128), lambda i: (0, i, 0)),
            pl.BlockSpec((_BP, 128), lambda i: (i, 0)),
            pl.BlockSpec((_BP, 128), lambda i: (i, 0)),
            pl.BlockSpec((1, 128), lambda i: (0, 0)),
        ],
        out_specs=pl.BlockSpec((_BP, 128), lambda i: (i, 0)),
        out_shape=jax.ShapeDtypeStruct((PR, 128), jnp.float32),
    )(accp, h2p, disp, b2rep)


def kernel(x, edge_index, W1, b1, W2, b2):
    ei4 = edge_index.reshape(2, NW, CH, CB)
    eye8 = jnp.eye(8, dtype=jnp.float32)
    w1_big = jnp.kron(eye8, W1)
    w2p = jnp.pad(W2, ((0, 0), (0, FW - F_OUT)))
    w2_big = jnp.kron(eye8, w2p)
    b1rep = jnp.tile(b1, 8).reshape(1, 128)
    b2rep = jnp.tile(jnp.pad(b2, (0, FW - F_OUT)), 8).reshape(1, 128)
    x_r = jnp.pad(x, ((0, NPAD - N), (0, 0))).reshape(PR, F_IN * 8)

    degp = _deg(ei4).reshape(NC, PR, 128)
    h1p, disp = _k1(x_r, w1_big, degp)

    acc1p = _agg(h1p.reshape(NPAD, FW), ei4).reshape(NC, PR, 128)
    h2p = _k3(acc1p, h1p, disp, w2_big, b1rep)
    acc2p = _agg(h2p.reshape(NPAD, FW), ei4).reshape(NC, PR, 128)
    outp = _k5(acc2p, h2p, disp, b2rep)
    return outp.reshape(NPAD, FW)[:N, :F_OUT]


# speedup vs baseline: 1.0416x; 1.0416x over previous
import functools

import jax
import jax.numpy as jnp
from jax import lax
from jax.experimental import pallas as pl
from jax.experimental.pallas import tpu as pltpu
from jax.experimental.pallas import tpu_sc as plsc

N = 10000
NPAD = 10240
F_IN = 128
HID = 16
FW = 16
F_OUT = 8
PR = NPAD * FW // 128

NC = 2
NS = 16
NW = NC * NS
CB = 125
CH = 80
NB = 20
NR = CH // NB
RPT = NPAD // NS

_mesh = plsc.VectorSubcoreMesh(
    core_axis_name="c", subcore_axis_name="s", num_cores=NC, num_subcores=NS
)
_sc_params = pltpu.CompilerParams(use_tc_tiling_on_sc=False)


@functools.partial(
    pl.kernel,
    mesh=_mesh,
    out_type=jax.ShapeDtypeStruct((NC, NPAD, FW), jnp.float32),
    scratch_types=[
        pltpu.VMEM((CH, CB), jnp.int32),
        pltpu.VMEM((128,), jnp.float32),
        pltpu.VMEM((RPT,), jnp.float32),
        pltpu.VMEM((RPT, FW), jnp.float32),
        pltpu.VMEM_SHARED((NPAD,), jnp.float32),
        pltpu.SemaphoreType.DMA,
    ],
    compiler_params=_sc_params,
)
def _deg(ei_hbm, out_hbm, dst_v, ones_v, st_v, buf_v, acc, sem):
    c = lax.axis_index("c")
    s = lax.axis_index("s")
    wid = s * NC + c
    dslab = pltpu.async_copy(ei_hbm.at[1, wid], dst_v, sem)
    for i in range(8):
        ones_v[pl.ds(i * 16, 16)] = jnp.ones((16,), jnp.float32)
    for i in range(RPT // 16):
        st_v[pl.ds(i * 16, 16)] = jnp.zeros((16,), jnp.float32)
    pltpu.sync_copy(st_v, acc.at[pl.ds(s * RPT, RPT)])
    dslab.wait()
    plsc.subcore_barrier()

    ones_row = ones_v.at[pl.ds(0, CB)]
    descs = [
        pltpu.async_copy(ones_row, acc.at[dst_v.at[j]], sem, add=True)
        for j in range(CH)
    ]
    for d in descs:
        d.wait()
    plsc.subcore_barrier()
    pltpu.sync_copy(acc.at[pl.ds(s * RPT, RPT)], st_v)

    def rep(g, carry):
        v = st_v[pl.ds(g * 16, 16)]
        for l in range(16):
            buf_v[g * 16 + l] = jnp.broadcast_to(v[l], (FW,))
        return carry

    lax.fori_loop(0, RPT // 16, rep, 0)
    pltpu.sync_copy(buf_v, out_hbm.at[c, pl.ds(s * RPT, RPT)])


@functools.partial(
    pl.kernel,
    mesh=_mesh,
    out_type=jax.ShapeDtypeStruct((NC, NPAD, FW), jnp.float32),
    scratch_types=[
        pltpu.VMEM((CH, CB), jnp.int32),
        pltpu.VMEM((CH, CB), jnp.int32),
        pltpu.VMEM((2 * NB, CB, FW), jnp.float32),
        pltpu.VMEM((RPT, FW), jnp.float32),
        pltpu.VMEM_SHARED((NPAD, FW), jnp.float32),
        pltpu.SemaphoreType.DMA,
        pltpu.SemaphoreType.DMA,
    ],
    compiler_params=_sc_params,
)
def _agg(tbl_hbm, ei_hbm, out_hbm,
         src_v, dst_v, rows_v, buf_v, acc, gsem, ssem):
    c = lax.axis_index("c")
    s = lax.axis_index("s")
    wid = s * NC + c
    sslab = pltpu.async_copy(ei_hbm.at[0, wid], src_v, gsem)
    dslab = pltpu.async_copy(ei_hbm.at[1, wid], dst_v, gsem)

    def zb(r, carry):
        buf_v[r] = jnp.zeros((FW,), jnp.float32)
        return carry

    lax.fori_loop(0, RPT, zb, 0)
    pltpu.sync_copy(buf_v, acc.at[pl.ds(s * RPT, RPT)])
    sslab.wait()
    dslab.wait()
    plsc.subcore_barrier()

    def gfire(r):
        bank = (r % 2) * NB
        j0 = r * NB
        return [
            pltpu.async_copy(tbl_hbm.at[src_v.at[j0 + b]],
                             rows_v.at[bank + b], gsem)
            for b in range(NB)
        ]

    gds = {0: gfire(0), 1: gfire(1)}
    for r in range(NR):
        bank = (r % 2) * NB
        j0 = r * NB
        sds = []
        for b in range(NB):
            gds[r][b].wait()
            sds.append(
                pltpu.async_copy(rows_v.at[bank + b], acc.at[dst_v.at[j0 + b]],
                                 ssem, add=True))
        for d in sds:
            d.wait()
        if r + 2 < NR:
            gds[r + 2] = gfire(r + 2)
    plsc.subcore_barrier()
    pltpu.sync_copy(acc.at[pl.ds(s * RPT, RPT)], buf_v)
    pltpu.sync_copy(buf_v, out_hbm.at[c, pl.ds(s * RPT, RPT)])


_BP = 128
_BE = 256


def _k1a_body(x_ref, w_ref, m_ref):
    m_ref[...] = jnp.dot(x_ref[...], w_ref[...],
                         preferred_element_type=jnp.float32)


def _k1a(x_r, w_big):
    return pl.pallas_call(
        _k1a_body,
        grid=(PR // _BP,),
        in_specs=[
            pl.BlockSpec((_BP, F_IN * 8), lambda i: (i, 0)),
            pl.BlockSpec((F_IN * 8, 128), lambda i: (0, 0)),
        ],
        out_specs=pl.BlockSpec((_BP, 128), lambda i: (i, 0)),
        out_shape=jax.ShapeDtypeStruct((PR, 128), jnp.float32),
    )(x_r, w_big)


def _k1b_body(m_ref, degp_ref, h_ref, dis_ref):
    deg = degp_ref[0] + degp_ref[1] + 1.0
    dis = lax.rsqrt(deg)
    h_ref[...] = m_ref[...] * dis
    dis_ref[...] = dis


def _k1b(m, degp):
    return pl.pallas_call(
        _k1b_body,
        grid=(PR // _BE,),
        in_specs=[
            pl.BlockSpec((_BE, 128), lambda i: (i, 0)),
            pl.BlockSpec((NC, _BE, 128), lambda i: (0, i, 0)),
        ],
        out_specs=[
            pl.BlockSpec((_BE, 128), lambda i: (i, 0)),
            pl.BlockSpec((_BE, 128), lambda i: (i, 0)),
        ],
        out_shape=[
            jax.ShapeDtypeStruct((PR, 128), jnp.float32),
            jax.ShapeDtypeStruct((PR, 128), jnp.float32),
        ],
    )(m, degp)


def _k3_body(a_ref, h1_ref, dis_ref, w_ref, b_ref, h_ref):
    dis = dis_ref[...]
    agg = a_ref[0] + a_ref[1] + h1_ref[...]
    z = jnp.maximum(dis * agg + b_ref[...], 0.0)
    h_ref[...] = dis * jnp.dot(z, w_ref[...],
                               preferred_element_type=jnp.float32)


def _k3(accp, h1p, disp, w2_big, b1rep):
    return pl.pallas_call(
        _k3_body,
        grid=(PR // _BE,),
        in_specs=[
            pl.BlockSpec((NC, _BE, 128), lambda i: (0, i, 0)),
            pl.BlockSpec((_BE, 128), lambda i: (i, 0)),
            pl.BlockSpec((_BE, 128), lambda i: (i, 0)),
            pl.BlockSpec((128, 128), lambda i: (0, 0)),
            pl.BlockSpec((1, 128), lambda i: (0, 0)),
        ],
        out_specs=pl.BlockSpec((_BE, 128), lambda i: (i, 0)),
        out_shape=jax.ShapeDtypeStruct((PR, 128), jnp.float32),
    )(accp, h1p, disp, w2_big, b1rep)


def _k5_body(a_ref, h2_ref, dis_ref, b_ref, o_ref):
    agg = a_ref[0] + a_ref[1] + h2_ref[...]
    o_ref[...] = dis_ref[...] * agg + b_ref[...]


def _k5(accp, h2p, disp, b2rep):
    return pl.pallas_call(
        _k5_body,
        grid=(PR // _BE,),
        in_specs=[
            pl.BlockSpec((NC, _BE, 128), lambda i: (0, i, 0)),
            pl.BlockSpec((_BE, 128), lambda i: (i, 0)),
            pl.BlockSpec((_BE, 128), lambda i: (i, 0)),
            pl.BlockSpec((1, 128), lambda i: (0, 0)),
        ],
        out_specs=pl.BlockSpec((_BE, 128), lambda i: (i, 0)),
        out_shape=jax.ShapeDtypeStruct((PR, 128), jnp.float32),
    )(accp, h2p, disp, b2rep)


def kernel(x, edge_index, W1, b1, W2, b2):
    ei4 = edge_index.reshape(2, NW, CH, CB)
    eye8 = jnp.eye(8, dtype=jnp.float32)
    w1_big = jnp.kron(eye8, W1)
    w2p = jnp.pad(W2, ((0, 0), (0, FW - F_OUT)))
    w2_big = jnp.kron(eye8, w2p)
    b1rep = jnp.tile(b1, 8).reshape(1, 128)
    b2rep = jnp.tile(jnp.pad(b2, (0, FW - F_OUT)), 8).reshape(1, 128)
    x_r = jnp.pad(x, ((0, NPAD - N), (0, 0))).reshape(PR, F_IN * 8)

    degp = _deg(ei4).reshape(NC, PR, 128)
    m1 = _k1a(x_r, w1_big)
    h1p, disp = _k1b(m1, degp)

    acc1p = _agg(h1p.reshape(NPAD, FW), ei4).reshape(NC, PR, 128)
    h2p = _k3(acc1p, h1p, disp, w2_big, b1rep)
    acc2p = _agg(h2p.reshape(NPAD, FW), ei4).reshape(NC, PR, 128)
    outp = _k5(acc2p, h2p, disp, b2rep)
    out16 = lax.slice(outp, (0, 0), (N * FW // 128, 128)).reshape(N, FW)
    return out16[:, :F_OUT]
